# Initial kernel scaffold; baseline (speedup 1.0000x reference)
import jax
import jax.numpy as jnp
from jax.experimental import pallas as pl


def kernel(u, V, A, params):
    raise NotImplementedError("write your pallas kernel here")


# speedup vs baseline: 2.1004x; 2.1004x over previous
import jax
import jax.numpy as jnp
from jax.experimental import pallas as pl
from jax.experimental.pallas import tpu as pltpu

B, N = 8, 256
GD, ND, ED = 32, 128, 16
NL = 3
TI = 64
NBLK = N // TI
PACK = N * ED // 128
PR = N * PACK


def _make_body(L):
    f32 = jnp.float32

    def dot(a, b):
        return jax.lax.dot_general(a, b, (((1,), (0,)), ((), ())),
                                   preferred_element_type=f32)

    def body(*args):
        k = 0

        def take(n):
            nonlocal k
            out = args[k:k + n]
            k += n
            return out

        Ap = take(L)
        Vs = take(L)
        us = take(L)
        BD = take(L)
        WeVi = take(L)
        WeVj = take(L)
        Weu = take(L)
        (be,) = take(1)
        WvV = take(L)
        (Wve,) = take(1)
        Wvu = take(L)
        (bv,) = take(1)
        Wuu = take(L)
        (WuV,) = take(1)
        (Wue,) = take(1)
        (bu,) = take(1)
        Aout, Vout, uout = take(3)
        Qb, Pb, rb, sV, sA = take(5)

        t = pl.program_id(1)

        @pl.when(t == 0)
        def _init():
            Q = dot(Vs[0][0], WeVj[0][...])
            P = dot(Vs[0][0], WeVi[0][...])
            r = dot(us[0][0], Weu[0][...])
            for s in range(1, L):
                Q = Q + dot(Vs[s][0], WeVj[s][...])
                P = P + dot(Vs[s][0], WeVi[s][...])
                r = r + dot(us[s][0], Weu[s][...])
            gi = jax.lax.broadcasted_iota(jnp.int32, (PACK, N), 0)
            ri = jax.lax.broadcasted_iota(jnp.int32, (PACK, N), 1)
            ci = jax.lax.broadcasted_iota(jnp.int32, (ED, 128), 0)
            li = jax.lax.broadcasted_iota(jnp.int32, (ED, 128), 1)
            Qp = jnp.zeros((PACK, 128), f32)
            for kk in range(128 // ED):
                Gk = (ri == (128 // ED) * gi + kk).astype(f32)
                Ek = (li == ED * kk + ci).astype(f32)
                Qp = Qp + dot(dot(Gk, Q), Ek)
            Qb[...] = Qp
            Pb[...] = P
            rb[...] = r + be[...]
            sV[...] = jnp.zeros((1, ND), f32)
            sA[...] = jnp.zeros((1, ED), f32)

        Y = dot(Ap[0][0], BD[0][...])
        for s in range(1, L):
            Y = Y + dot(Ap[s][0], BD[s][...])
        Pblk = Pb[pl.ds(t * TI, TI), :] + rb[...]
        Pt = jnp.tile(Pblk, (1, 128 // ED))
        Ppack = jnp.broadcast_to(Pt[:, None, :], (TI, PACK, 128))
        Ppack = Ppack.reshape(TI * PACK, 128)
        Qpack = jnp.broadcast_to(Qb[...][None, :, :], (TI, PACK, 128))
        Qpack = Qpack.reshape(TI * PACK, 128)
        Z = jnp.maximum(Y + Ppack + Qpack, 0.0)
        Aout[0] = Z

        li = jax.lax.broadcasted_iota(jnp.int32, (128, ED), 0)
        ci = jax.lax.broadcasted_iota(jnp.int32, (128, ED), 1)
        S = (li % ED == ci).astype(f32)
        aggS = dot(Z, S)
        agg3 = aggS.reshape(TI, PACK, ED).sum(axis=1)
        agge = agg3 * (1.0 / N)

        Vacc = dot(agge, Wve[...]) + bv[...]
        for s in range(L):
            Vacc = Vacc + dot(Vs[s][0, pl.ds(t * TI, TI), :], WvV[s][...])
            Vacc = Vacc + dot(us[s][0], Wvu[s][...])
        Vb = jnp.maximum(Vacc, 0.0)
        Vout[0] = Vb

        sV[...] += Vb.sum(axis=0, keepdims=True)
        sA[...] += agg3.sum(axis=0, keepdims=True)

        @pl.when(t == NBLK - 1)
        def _fin():
            ua = dot(sV[...] * (1.0 / N), WuV[...])
            ua = ua + dot(sA[...] * (1.0 / (N * N)), Wue[...])
            ua = ua + bu[...]
            for s in range(L):
                ua = ua + dot(us[s][0], Wuu[s][...])
            uout[0] = jnp.maximum(ua, 0.0)

    return body


def _layer(L, Aps, Vls, uls, We, be, Wv, bv, Wu, bu):
    f32 = jnp.float32
    ie, iv, ig = ED * L, ND * L, GD * L
    eye8 = jnp.eye(128 // ED, dtype=f32)
    BDs = [jnp.kron(eye8, We[ED * s: ED * (s + 1), :]) for s in range(L)]
    WeVi = [We[ie + ND * s: ie + ND * (s + 1)] for s in range(L)]
    WeVj = [We[ie + iv + ND * s: ie + iv + ND * (s + 1)] for s in range(L)]
    Weu = [We[ie + 2 * iv + GD * s: ie + 2 * iv + GD * (s + 1)] for s in range(L)]
    WvVs = [Wv[ND * s: ND * (s + 1)] for s in range(L)]
    Wve = Wv[iv: iv + ED]
    Wvus = [Wv[iv + ED + GD * s: iv + ED + GD * (s + 1)] for s in range(L)]
    Wuus = [Wu[GD * s: GD * (s + 1)] for s in range(L)]
    WuV = Wu[ig: ig + ND]
    Wue = Wu[ig + ND: ig + ND + ED]

    weights = (BDs + WeVi + WeVj + Weu + [be[None]] + WvVs + [Wve] + Wvus
               + [bv[None]] + Wuus + [WuV] + [Wue] + [bu[None]])

    in_specs = (
        [pl.BlockSpec((1, TI * PACK, 128), lambda b, t: (b, t, 0))] * L
        + [pl.BlockSpec((1, N, ND), lambda b, t: (b, 0, 0))] * L
        + [pl.BlockSpec((1, 1, GD), lambda b, t: (b, 0, 0))] * L
        + [pl.BlockSpec(w.shape, lambda b, t, nd=w.ndim: (0,) * nd)
           for w in weights]
    )
    out_specs = [
        pl.BlockSpec((1, TI * PACK, 128), lambda b, t: (b, t, 0)),
        pl.BlockSpec((1, TI, ND), lambda b, t: (b, t, 0)),
        pl.BlockSpec((1, 1, GD), lambda b, t: (b, 0, 0)),
    ]
    out_shape = [
        jax.ShapeDtypeStruct((B, PR, 128), f32),
        jax.ShapeDtypeStruct((B, N, ND), f32),
        jax.ShapeDtypeStruct((B, 1, GD), f32),
    ]
    scratch = [
        pltpu.VMEM((PACK, 128), f32),
        pltpu.VMEM((N, ED), f32),
        pltpu.VMEM((1, ED), f32),
        pltpu.VMEM((1, ND), f32),
        pltpu.VMEM((1, ED), f32),
    ]

    Apk, Vpk, upk = _layer_call(L)(
        *Aps, *Vls, *uls, *weights,
        in_specs=in_specs, out_specs=out_specs, out_shape=out_shape,
        scratch=scratch)
    return upk, Vpk, Apk


def _layer_call(L):
    def run(*ops, in_specs, out_specs, out_shape, scratch):
        return pl.pallas_call(
            _make_body(L),
            grid=(B, NBLK),
            in_specs=in_specs,
            out_specs=out_specs,
            out_shape=out_shape,
            scratch_shapes=scratch,
            compiler_params=pltpu.CompilerParams(
                dimension_semantics=("parallel", "arbitrary")),
        )(*ops)
    return run


def kernel(u, V, A, params):
    f32 = jnp.float32
    Aps = [A.astype(f32).reshape(B, PR, 128)]
    Vls = [V.astype(f32)]
    uls = [u.astype(f32).reshape(B, 1, GD)]
    u_ = V_ = None
    for i in range(NL):
        We, be, Wv, bv, Wu, bu = params[6 * i: 6 * (i + 1)]
        u_, V_, Ap_ = _layer(i + 1, Aps, Vls, uls, We, be, Wv, bv, Wu, bu)
        Aps.append(Ap_)
        Vls.append(V_)
        uls.append(u_)
    return (u_.reshape(B, GD), V_, Aps[-1].reshape(B, N, N, ED))


# speedup vs baseline: 2.4726x; 1.1772x over previous
import jax
import jax.numpy as jnp
from jax.experimental import pallas as pl
from jax.experimental.pallas import tpu as pltpu

B, N = 8, 256
GD, ND, ED = 32, 128, 16
NL = 3
EPR = 128 // ED
PACK = N // EPR
PR = N * PACK
NCH = 4
CI = N // NCH
CR = PR // NCH

_f32 = jnp.float32


def _dot(a, b):
    return jax.lax.dot_general(a, b, (((1,), (0,)), ((), ())),
                               preferred_element_type=_f32)


def _body(*args):
    k = 0

    def take(n):
        nonlocal k
        out = args[k:k + n]
        k += n
        return out

    (A0,) = take(1)
    (V0,) = take(1)
    (u0,) = take(1)
    W = []
    for l in range(1, NL + 1):
        BD = take(l)
        WeVi = take(l)
        WeVj = take(l)
        Weu = take(l)
        (be,) = take(1)
        WvV = take(l)
        (Wve,) = take(1)
        Wvu = take(l)
        (bv,) = take(1)
        Wuu = take(l)
        (WuV,) = take(1)
        (Wue,) = take(1)
        (bu,) = take(1)
        W.append((BD, WeVi, WeVj, Weu, be, WvV, Wve, Wvu, bv,
                  Wuu, WuV, Wue, bu))
    A3, V3, u3 = take(3)
    A1s, A2s = take(2)

    li = jax.lax.broadcasted_iota(jnp.int32, (128, ED), 0)
    ci = jax.lax.broadcasted_iota(jnp.int32, (128, ED), 1)
    S = (li % ED == ci).astype(_f32)

    Vv = [V0[0]]
    uv = [u0[0]]

    def a_chunk(s, c):
        if s == 0:
            return A0[0, pl.ds(c * CR, CR), :]
        return (A1s, A2s)[s - 1][pl.ds(c * CR, CR), :]

    def a_store(l, c, Z):
        if l == NL:
            A3[0, pl.ds(c * CR, CR), :] = Z
        else:
            (A1s, A2s)[l - 1][pl.ds(c * CR, CR), :] = Z

    for l in range(1, NL + 1):
        (BD, WeVi, WeVj, Weu, be, WvV, Wve, Wvu, bv,
         Wuu, WuV, Wue, bu) = W[l - 1]

        P = _dot(Vv[0], WeVi[0][...])
        Q = _dot(Vv[0], WeVj[0][...])
        r = _dot(uv[0], Weu[0][...])
        for s in range(1, l):
            P = P + _dot(Vv[s], WeVi[s][...])
            Q = Q + _dot(Vv[s], WeVj[s][...])
            r = r + _dot(uv[s], Weu[s][...])
        Pr = P + (r + be[...])

        gi = jax.lax.broadcasted_iota(jnp.int32, (PACK, N), 0)
        ri = jax.lax.broadcasted_iota(jnp.int32, (PACK, N), 1)
        qci = jax.lax.broadcasted_iota(jnp.int32, (ED, 128), 0)
        qli = jax.lax.broadcasted_iota(jnp.int32, (ED, 128), 1)
        Qp = jnp.zeros((PACK, 128), _f32)
        for kk in range(EPR):
            Gk = (ri == EPR * gi + kk).astype(_f32)
            Ek = (qli == ED * kk + qci).astype(_f32)
            Qp = Qp + _dot(_dot(Gk, Q), Ek)
        Qpack = jnp.broadcast_to(Qp[None, :, :], (CI, PACK, 128))
        Qpack = Qpack.reshape(CR, 128)

        aggs = []
        for c in range(NCH):
            Y = _dot(a_chunk(0, c), BD[0][...])
            for s in range(1, l):
                Y = Y + _dot(a_chunk(s, c), BD[s][...])
            Pblk = Pr[c * CI:(c + 1) * CI, :]
            Pt = jnp.tile(Pblk, (1, EPR))
            Ppack = jnp.broadcast_to(Pt[:, None, :], (CI, PACK, 128))
            Ppack = Ppack.reshape(CR, 128)
            Z = jnp.maximum(Y + Ppack + Qpack, 0.0)
            a_store(l, c, Z)
            aggS = _dot(Z, S)
            aggs.append(aggS.reshape(CI, PACK, ED).sum(axis=1))
        agg = jnp.concatenate(aggs, axis=0)
        sumA = agg.sum(axis=0, keepdims=True)
        agge = agg * (1.0 / N)

        Vacc = _dot(agge, Wve[...]) + bv[...]
        for s in range(l):
            Vacc = Vacc + _dot(Vv[s], WvV[s][...])
            Vacc = Vacc + _dot(uv[s], Wvu[s][...])
        Vl = jnp.maximum(Vacc, 0.0)

        ua = _dot(Vl.sum(axis=0, keepdims=True) * (1.0 / N), WuV[...])
        ua = ua + _dot(sumA * (1.0 / (N * N)), Wue[...]) + bu[...]
        for s in range(l):
            ua = ua + _dot(uv[s], Wuu[s][...])
        ul = jnp.maximum(ua, 0.0)

        Vv.append(Vl)
        uv.append(ul)

    V3[0] = Vv[NL]
    u3[0] = uv[NL]


def kernel(u, V, A, params):
    Ap = A.astype(_f32).reshape(B, PR, 128)
    V0 = V.astype(_f32)
    u0 = u.astype(_f32).reshape(B, 1, GD)

    eye8 = jnp.eye(EPR, dtype=_f32)
    weights = []
    for i in range(NL):
        l = i + 1
        We, be, Wv, bv, Wu, bu = params[6 * i: 6 * (i + 1)]
        ie, iv, ig = ED * l, ND * l, GD * l
        weights += [jnp.kron(eye8, We[ED * s: ED * (s + 1), :])
                    for s in range(l)]
        weights += [We[ie + ND * s: ie + ND * (s + 1)] for s in range(l)]
        weights += [We[ie + iv + ND * s: ie + iv + ND * (s + 1)]
                    for s in range(l)]
        weights += [We[ie + 2 * iv + GD * s: ie + 2 * iv + GD * (s + 1)]
                    for s in range(l)]
        weights += [be[None]]
        weights += [Wv[ND * s: ND * (s + 1)] for s in range(l)]
        weights += [Wv[iv: iv + ED]]
        weights += [Wv[iv + ED + GD * s: iv + ED + GD * (s + 1)]
                    for s in range(l)]
        weights += [bv[None]]
        weights += [Wu[GD * s: GD * (s + 1)] for s in range(l)]
        weights += [Wu[ig: ig + ND]]
        weights += [Wu[ig + ND: ig + ND + ED]]
        weights += [bu[None]]

    in_specs = (
        [pl.BlockSpec((1, PR, 128), lambda b: (b, 0, 0)),
         pl.BlockSpec((1, N, ND), lambda b: (b, 0, 0)),
         pl.BlockSpec((1, 1, GD), lambda b: (b, 0, 0))]
        + [pl.BlockSpec(w.shape, lambda b, nd=w.ndim: (0,) * nd)
           for w in weights]
    )
    out_specs = [
        pl.BlockSpec((1, PR, 128), lambda b: (b, 0, 0)),
        pl.BlockSpec((1, N, ND), lambda b: (b, 0, 0)),
        pl.BlockSpec((1, 1, GD), lambda b: (b, 0, 0)),
    ]
    out_shape = [
        jax.ShapeDtypeStruct((B, PR, 128), _f32),
        jax.ShapeDtypeStruct((B, N, ND), _f32),
        jax.ShapeDtypeStruct((B, 1, GD), _f32),
    ]
    scratch = [
        pltpu.VMEM((PR, 128), _f32),
        pltpu.VMEM((PR, 128), _f32),
    ]

    A3, V3, u3 = pl.pallas_call(
        _body,
        grid=(B,),
        in_specs=in_specs,
        out_specs=out_specs,
        out_shape=out_shape,
        scratch_shapes=scratch,
        compiler_params=pltpu.CompilerParams(
            dimension_semantics=("arbitrary",)),
    )(Ap, V0, u0, *weights)

    return (u3.reshape(B, GD), V3, A3.reshape(B, N, N, ED))
